# baseline (speedup 1.0000x reference)
import functools

import jax
import jax.numpy as jnp
from jax import lax
from jax.experimental import pallas as pl
from jax.experimental.pallas import tpu as pltpu

F32 = jnp.float32
BF16 = jnp.bfloat16
HI = lax.Precision.HIGHEST


def _dot(a, b, dims, prec=None):
    return lax.dot_general(a, b, (dims, ((), ())), precision=prec,
                           preferred_element_type=F32)


def _xe_body(f_ref, w_ref, o_ref):
    o_ref[0] = _dot(f_ref[0], w_ref[0], ((1,), (0,)), HI)


def _enc_body(adj_ref, gn_ref, f_ref, lbl_ref, xe_ref, xea_ref,
              dw_ref, cw_ref, b_ref, h_ref, sn_ref, sl_ref, ft_ref,
              adj_scr, h_scr, hr_scr, har_scr, *, br):
    k = pl.program_id(0)
    ph = pl.program_id(1)
    i = pl.program_id(2)
    rows = pl.dslice(i * br, br)

    @pl.when(ph == 0)
    def _phase0():
        a = adj_ref[0].astype(BF16)
        adj_scr[rows, :] = a
        h = _dot(a, xe_ref[0].astype(BF16), ((1,), (0,)))
        ha = _dot(a, xea_ref[0].astype(BF16), ((1,), (0,)))
        h_ref[0] = h
        hr = jnp.maximum(h, 0.0)
        har = jnp.maximum(ha, 0.0)
        h_scr[rows, :] = h.astype(BF16)
        hr_scr[rows, :] = hr.astype(BF16)
        har_scr[rows, :] = har.astype(BF16)
        sn_ref[0, 0] = jnp.sum(h * h, axis=1)

    @pl.when(ph == 1)
    def _phase1():
        @pl.when((k == 0) & (i == 0))
        def _init():
            sl_ref[0, 0] = jnp.float32(0.0)
            ft_ref[0, 0] = jnp.float32(0.0)

        g32 = gn_ref[0]
        gb = g32.astype(BF16)
        ah = _dot(adj_scr[rows, :], h_scr[...], ((1,), (0,)))
        vs = _dot(gb, hr_scr[...], ((1,), (0,)))
        vsa = _dot(gb, har_scr[...], ((1,), (0,)))
        rs = jnp.sum(g32, axis=1)

        def readout(v):
            g = v / rs[:, None]
            nrm = jnp.sqrt(jnp.sum(g * g, axis=1, keepdims=True))
            g = g / jnp.maximum(nrm, 1e-12)
            return jax.nn.sigmoid(g)

        g = readout(vs)
        ga = readout(vsa)
        cwb = cw_ref[...].astype(BF16)
        t = _dot(hr_scr[rows, :], cwb, ((1,), (0,)))
        ta = _dot(har_scr[rows, :], cwb, ((1,), (0,)))
        b = b_ref[0, 0]
        sc1 = jnp.sum(t * g, axis=1) + b
        sc2 = jnp.sum(ta * g, axis=1) + b
        sc1a = jnp.sum(ta * ga, axis=1) + b
        sc2a = jnp.sum(t * ga, axis=1) + b
        l0 = lbl_ref[0, :, 0]
        l1 = lbl_ref[0, :, 1]

        def bce(x, tt):
            return jnp.sum(jnp.maximum(x, 0.0) - x * tt
                           + jnp.log1p(jnp.exp(-jnp.abs(x))))

        sl_ref[0, 0] += (bce(sc1, l0) + bce(sc2, l1)
                         + bce(sc1a, l0) + bce(sc2a, l1))
        out = _dot(ah.astype(BF16), dw_ref[0].astype(BF16), ((1,), (0,)))
        d = f_ref[0] - out
        ft_ref[0, 0] += jnp.sum(d * d)


def _sink_body(m_ref, h0_ref, h1_ref, sn0_ref, sn1_ref,
               cxy_ref, cyxt_ref, align_ref, spars_ref, marg_ref, bad_ref,
               a_scr, cs_scr, g_scr, *, nn, lt, brp):
    p = pl.program_id(0)
    i = pl.program_id(1)
    rows = pl.dslice(i * brp, brp)

    @pl.when(p == 0)
    def _p0():
        @pl.when(i == 0)
        def _():
            cs_scr[0:1] = jnp.zeros_like(cs_scr[0:1])
            mm = _dot(m_ref[...], m_ref[...], ((1,), (1,)), HI)
            g_scr[...] = _dot(h1_ref[0], mm, ((1,), (0,)), HI)

        c = _dot(h0_ref[0], g_scr[...], ((1,), (1,)), HI)
        c = c * (1.0 / jnp.sqrt(jnp.float32(lt)))
        m = jnp.max(c, axis=1, keepdims=True)
        e = jnp.exp(c - m)
        s = jnp.sum(e, axis=1, keepdims=True)
        a = e * (1.0 / (s * nn))
        rs = jnp.sum(a, axis=1, keepdims=True)
        a = a * (1.0 / (rs + 1e-8))
        a_scr[rows, :] = a.astype(BF16)
        cs_scr[0:1] += jnp.sum(a, axis=0, keepdims=True)

    def _mid(r):
        @pl.when(i == 0)
        def _():
            cs_scr[r + 1:r + 2] = jnp.zeros_like(cs_scr[0:1])

        a = a_scr[rows, :].astype(F32) * (1.0 / (cs_scr[r:r + 1] + 1e-8))
        rs = jnp.sum(a, axis=1, keepdims=True)
        a = a * (1.0 / (rs + 1e-8))
        a_scr[rows, :] = a.astype(BF16)
        cs_scr[r + 1:r + 2] += jnp.sum(a, axis=0, keepdims=True)

    @pl.when(p == 1)
    def _p1():
        _mid(0)

    @pl.when(p == 2)
    def _p2():
        _mid(1)

    @pl.when(p == 3)
    def _p3():
        cs3 = cs_scr[2:3]

        @pl.when(i == 0)
        def _():
            align_ref[0, 0] = jnp.float32(0.0)
            spars_ref[0, 0] = jnp.float32(0.0)
            c3 = cs3[0]
            q = c3 / (c3 + 1e-8)
            marg_ref[0, 0] = jnp.sum(q * (jnp.log(q + 1e-12) - (1.0 / nn)))
            bad_ref[0, 0] = jnp.sum((c3 == 0.0).astype(F32))

        a = a_scr[rows, :].astype(F32) * (1.0 / (cs3 + 1e-8))
        rsa = jnp.sum(a, axis=1, keepdims=True)
        cxy_ref[...] = (a * (1.0 / rsa)).astype(BF16)
        csa = cs3 / (cs3 + 1e-8)
        cyxt_ref[...] = (a * (1.0 / csa)).astype(BF16)
        bad_ref[0, 0] += jnp.sum((rsa == 0.0).astype(F32))
        d2 = (sn0_ref[0, 0][:, None] + sn1_ref[0, 0][None, :]
              - 2.0 * _dot(h0_ref[0], h1_ref[0], ((1,), (1,)), HI))
        cd = jnp.sqrt(jnp.maximum(d2, 0.0))
        align_ref[0, 0] += jnp.sum(a * cd)
        spars_ref[0, 0] += jnp.sum(a * jnp.log(a + 1e-10))


def _t1_body(d_ref, c_ref, o_ref):
    o_ref[...] = _dot(d_ref[...].astype(BF16), c_ref[...],
                      ((1,), (1,))).astype(BF16)


def _t2_body(d_ref, c_ref, o_ref):
    o_ref[...] = _dot(d_ref[...].astype(BF16), c_ref[...],
                      ((1,), (0,))).astype(BF16)


def _p1_body(c_ref, t_ref, d0_ref, f1_ref, f0_ref, ssq_ref, mse_ref):
    @pl.when(pl.program_id(0) == 0)
    def _init():
        ssq_ref[0, 0] = jnp.float32(0.0)
        mse_ref[0, 0] = jnp.float32(0.0)

    p = _dot(c_ref[...], t_ref[...], ((1,), (0,)))
    d = d0_ref[...] - p
    ssq_ref[0, 0] += jnp.sum(d * d)
    e = f0_ref[...] - _dot(c_ref[...], f1_ref[...], ((1,), (0,)))
    mse_ref[0, 0] += jnp.sum(e * e)


def _p2_body(ct_ref, t_ref, d1_ref, f0_ref, f1_ref, ssq_ref, mse_ref):
    @pl.when(pl.program_id(0) == 0)
    def _init():
        ssq_ref[0, 0] = jnp.float32(0.0)
        mse_ref[0, 0] = jnp.float32(0.0)

    p = _dot(ct_ref[...], t_ref[...], ((0,), (0,)))
    d = d1_ref[...] - p
    ssq_ref[0, 0] += jnp.sum(d * d)
    e = f1_ref[...] - _dot(ct_ref[...], f0_ref[...], ((0,), (0,)))
    mse_ref[0, 0] += jnp.sum(e * e)


def _scalar_spec():
    return pl.BlockSpec(memory_space=pltpu.SMEM)


def _heavy_losses(cxy_bf, cyxt_bf, f0, f1, d0, d1):
    n, fd = f0.shape
    bm = min(512, n)
    grid = (n // bm,)
    f0_bf = f0.astype(BF16)
    f1_bf = f1.astype(BF16)

    t1 = pl.pallas_call(
        _t1_body, grid=grid,
        in_specs=[pl.BlockSpec((bm, n), lambda i: (i, 0)),
                  pl.BlockSpec((n, n), lambda i: (0, 0))],
        out_specs=pl.BlockSpec((bm, n), lambda i: (i, 0)),
        out_shape=jax.ShapeDtypeStruct((n, n), BF16))(d1, cxy_bf)

    ssq1, mse1 = pl.pallas_call(
        _p1_body, grid=grid,
        in_specs=[pl.BlockSpec((bm, n), lambda i: (i, 0)),
                  pl.BlockSpec((n, n), lambda i: (0, 0)),
                  pl.BlockSpec((bm, n), lambda i: (i, 0)),
                  pl.BlockSpec((n, fd), lambda i: (0, 0)),
                  pl.BlockSpec((bm, fd), lambda i: (i, 0))],
        out_specs=[_scalar_spec(), _scalar_spec()],
        out_shape=[jax.ShapeDtypeStruct((1, 1), F32),
                   jax.ShapeDtypeStruct((1, 1), F32)])(
        cxy_bf, t1, d0, f1_bf, f0)

    t2 = pl.pallas_call(
        _t2_body, grid=grid,
        in_specs=[pl.BlockSpec((bm, n), lambda i: (i, 0)),
                  pl.BlockSpec((n, n), lambda i: (0, 0))],
        out_specs=pl.BlockSpec((bm, n), lambda i: (i, 0)),
        out_shape=jax.ShapeDtypeStruct((n, n), BF16))(d0, cyxt_bf)

    ssq2, mse2 = pl.pallas_call(
        _p2_body, grid=grid,
        in_specs=[pl.BlockSpec((n, bm), lambda i: (0, i)),
                  pl.BlockSpec((n, n), lambda i: (0, 0)),
                  pl.BlockSpec((bm, n), lambda i: (i, 0)),
                  pl.BlockSpec((n, fd), lambda i: (0, 0)),
                  pl.BlockSpec((bm, fd), lambda i: (i, 0))],
        out_specs=[_scalar_spec(), _scalar_spec()],
        out_shape=[jax.ShapeDtypeStruct((1, 1), F32),
                   jax.ShapeDtypeStruct((1, 1), F32)])(
        cyxt_bf, t2, d1, f0_bf, f1)

    la_fix = (mse1[0, 0] + mse2[0, 0]) / jnp.float32(n * fd)
    l_maint = (jnp.sqrt(ssq1[0, 0]) + jnp.sqrt(ssq2[0, 0])) / jnp.float32(n)
    return la_fix, l_maint


def kernel(feature_set, spot_feature_set, adj_set, graph_neigh_set,
           label_CSL_set, dist_mat_set, enc_w, dec_w, disc_w, disc_b, Ms):
    del spot_feature_set
    kb, n, fd = feature_set.shape
    lt = enc_w.shape[-1]
    br = min(512, n)
    ni = n // br
    brp = min(512, n)
    nip = n // brp
    m0 = Ms[0]
    b2 = jnp.reshape(disc_b, (1, 1)).astype(F32)

    xe = pl.pallas_call(
        _xe_body, grid=(kb,),
        in_specs=[pl.BlockSpec((1, n, fd), lambda k: (k, 0, 0)),
                  pl.BlockSpec((1, fd, lt), lambda k: (k, 0, 0))],
        out_specs=pl.BlockSpec((1, n, lt), lambda k: (k, 0, 0)),
        out_shape=jax.ShapeDtypeStruct((kb, n, lt), F32))(feature_set, enc_w)
    perm = jax.random.permutation(jax.random.key(42), n)
    xea = jnp.take(xe, perm, axis=1)

    ni1 = ni - 1

    def _ph0_in(k, ph, i):
        return (k, jnp.where(ph == 0, i, ni1), 0)

    def _ph1_in(k, ph, i):
        return (k, jnp.where(ph == 1, i, 0), 0)

    h, sn, sl_raw, ft_raw = pl.pallas_call(
        functools.partial(_enc_body, br=br),
        grid=(kb, 2, ni),
        in_specs=[pl.BlockSpec((1, br, n), _ph0_in),
                  pl.BlockSpec((1, br, n), _ph1_in),
                  pl.BlockSpec((1, br, fd), _ph1_in),
                  pl.BlockSpec((1, br, 2), _ph1_in),
                  pl.BlockSpec((1, n, lt), lambda k, ph, i: (k, 0, 0)),
                  pl.BlockSpec((1, n, lt), lambda k, ph, i: (k, 0, 0)),
                  pl.BlockSpec((1, lt, fd), lambda k, ph, i: (k, 0, 0)),
                  pl.BlockSpec((lt, lt), lambda k, ph, i: (0, 0)),
                  _scalar_spec()],
        out_specs=[pl.BlockSpec((1, br, lt), _ph0_in),
                   pl.BlockSpec((1, 1, br),
                                lambda k, ph, i:
                                (k, 0, jnp.where(ph == 0, i, ni1))),
                   _scalar_spec(), _scalar_spec()],
        out_shape=[jax.ShapeDtypeStruct((kb, n, lt), F32),
                   jax.ShapeDtypeStruct((kb, 1, n), F32),
                   jax.ShapeDtypeStruct((1, 1), F32),
                   jax.ShapeDtypeStruct((1, 1), F32)],
        scratch_shapes=[pltpu.VMEM((n, n), BF16),
                        pltpu.VMEM((n, lt), BF16),
                        pltpu.VMEM((n, lt), BF16),
                        pltpu.VMEM((n, lt), BF16)])(
        adj_set, graph_neigh_set, feature_set, label_CSL_set, xe, xea,
        dec_w, disc_w, b2)

    def _out_map(p, i):
        return (jnp.where(p == 3, i, 0), 0)

    cxy_bf, cyxt_bf, align_raw, spars_raw, marg_raw, bad = pl.pallas_call(
        functools.partial(_sink_body, nn=float(n), lt=lt, brp=brp),
        grid=(4, nip),
        in_specs=[pl.BlockSpec((lt, lt), lambda p, i: (0, 0)),
                  pl.BlockSpec((1, brp, lt), lambda p, i: (0, i, 0)),
                  pl.BlockSpec((1, n, lt), lambda p, i: (1, 0, 0)),
                  pl.BlockSpec((1, 1, brp), lambda p, i: (0, 0, i)),
                  pl.BlockSpec((1, 1, n), lambda p, i: (1, 0, 0))],
        out_specs=[pl.BlockSpec((brp, n), _out_map),
                   pl.BlockSpec((brp, n), _out_map),
                   _scalar_spec(), _scalar_spec(), _scalar_spec(),
                   _scalar_spec()],
        out_shape=[jax.ShapeDtypeStruct((n, n), BF16),
                   jax.ShapeDtypeStruct((n, n), BF16),
                   jax.ShapeDtypeStruct((1, 1), F32),
                   jax.ShapeDtypeStruct((1, 1), F32),
                   jax.ShapeDtypeStruct((1, 1), F32),
                   jax.ShapeDtypeStruct((1, 1), F32)],
        scratch_shapes=[pltpu.VMEM((n, n), BF16),
                        pltpu.VMEM((8, n), F32),
                        pltpu.VMEM((n, lt), F32)])(
        m0, h, h, sn, sn)

    f0 = feature_set[0]
    f1 = feature_set[1]
    d0 = dist_mat_set[0]
    d1 = dist_mat_set[1]

    la_fix, l_maint = lax.cond(
        bad[0, 0] > 0.0,
        lambda ops: (jnp.float32(jnp.nan), jnp.float32(jnp.nan)),
        lambda ops: _heavy_losses(*ops),
        (cxy_bf, cyxt_bf, f0, f1, d0, d1))

    nf = jnp.float32(n)
    loss_sl = sl_raw[0, 0] / (nf * 2.0)
    loss_feat = ft_raw[0, 0] / (nf * jnp.float32(fd))
    loss_align = align_raw[0, 0]
    kl_pq = marg_raw[0, 0] / nf
    p = 1.0 / nf
    kl_pp = p * (jnp.log(p + 1e-12) - p)
    loss_marginal = (kl_pq - kl_pp) * nf
    loss_sparsity = -spars_raw[0, 0]
    return jnp.stack([loss_sl, loss_feat, loss_align, la_fix, l_maint,
                      loss_marginal, loss_sparsity])


# speedup vs baseline: 1.0372x; 1.0372x over previous
import functools

import jax
import jax.numpy as jnp
from jax import lax
from jax.experimental import pallas as pl
from jax.experimental.pallas import tpu as pltpu

F32 = jnp.float32
BF16 = jnp.bfloat16
HI = lax.Precision.HIGHEST


def _dot(a, b, dims, prec=None):
    return lax.dot_general(a, b, (dims, ((), ())), precision=prec,
                           preferred_element_type=F32)


def _xe_body(f_ref, w_ref, o_ref):
    o_ref[0] = _dot(f_ref[0], w_ref[0], ((1,), (0,)), HI)


def _enc_body(adj_ref, gn_ref, f_ref, lbl_ref, xe_ref, xea_ref,
              dw_ref, cw_ref, b_ref, h_ref, sn_ref, sl_ref, ft_ref,
              adj_scr, h_scr, hr_scr, har_scr, *, br):
    k = pl.program_id(0)
    ph = pl.program_id(1)
    i = pl.program_id(2)
    rows = pl.dslice(i * br, br)

    @pl.when(ph == 0)
    def _phase0():
        a = adj_ref[0].astype(BF16)
        adj_scr[rows, :] = a
        h = _dot(a, xe_ref[0].astype(BF16), ((1,), (0,)))
        ha = _dot(a, xea_ref[0].astype(BF16), ((1,), (0,)))
        h_ref[0] = h
        hr = jnp.maximum(h, 0.0)
        har = jnp.maximum(ha, 0.0)
        h_scr[rows, :] = h.astype(BF16)
        hr_scr[rows, :] = hr.astype(BF16)
        har_scr[rows, :] = har.astype(BF16)
        sn_ref[0, 0] = jnp.sum(h * h, axis=1)

    @pl.when(ph == 1)
    def _phase1():
        @pl.when((k == 0) & (i == 0))
        def _init():
            sl_ref[0, 0] = jnp.float32(0.0)
            ft_ref[0, 0] = jnp.float32(0.0)

        g32 = gn_ref[0]
        gb = g32.astype(BF16)
        ah = _dot(adj_scr[rows, :], h_scr[...], ((1,), (0,)))
        vs = _dot(gb, hr_scr[...], ((1,), (0,)))
        vsa = _dot(gb, har_scr[...], ((1,), (0,)))
        rs = jnp.sum(g32, axis=1)

        def readout(v):
            g = v / rs[:, None]
            nrm = jnp.sqrt(jnp.sum(g * g, axis=1, keepdims=True))
            g = g / jnp.maximum(nrm, 1e-12)
            return jax.nn.sigmoid(g)

        g = readout(vs)
        ga = readout(vsa)
        cwb = cw_ref[...].astype(BF16)
        t = _dot(hr_scr[rows, :], cwb, ((1,), (0,)))
        ta = _dot(har_scr[rows, :], cwb, ((1,), (0,)))
        b = b_ref[0, 0]
        sc1 = jnp.sum(t * g, axis=1) + b
        sc2 = jnp.sum(ta * g, axis=1) + b
        sc1a = jnp.sum(ta * ga, axis=1) + b
        sc2a = jnp.sum(t * ga, axis=1) + b
        l0 = lbl_ref[0, :, 0]
        l1 = lbl_ref[0, :, 1]

        def bce(x, tt):
            return jnp.sum(jnp.maximum(x, 0.0) - x * tt
                           + jnp.log1p(jnp.exp(-jnp.abs(x))))

        sl_ref[0, 0] += (bce(sc1, l0) + bce(sc2, l1)
                         + bce(sc1a, l0) + bce(sc2a, l1))
        out = _dot(ah.astype(BF16), dw_ref[0].astype(BF16), ((1,), (0,)))
        d = f_ref[0] - out
        ft_ref[0, 0] += jnp.sum(d * d)


def _sink_body(m_ref, h0_ref, h1_ref, sn0_ref, sn1_ref,
               cxy_ref, cyxt_ref, align_ref, spars_ref, marg_ref, bad_ref,
               a_scr, cs_scr, g_scr, *, nn, lt, brp):
    p = pl.program_id(0)
    i = pl.program_id(1)
    rows = pl.dslice(i * brp, brp)

    @pl.when(p == 0)
    def _p0():
        @pl.when(i == 0)
        def _():
            cs_scr[0:1] = jnp.zeros_like(cs_scr[0:1])
            mm = _dot(m_ref[...], m_ref[...], ((1,), (1,)), HI)
            g_scr[...] = _dot(h1_ref[0], mm, ((1,), (0,)), HI)

        c = _dot(h0_ref[0], g_scr[...], ((1,), (1,)), HI)
        c = c * (1.0 / jnp.sqrt(jnp.float32(lt)))
        m = jnp.max(c, axis=1, keepdims=True)
        e = jnp.exp(c - m)
        s = jnp.sum(e, axis=1, keepdims=True)
        a = e * (1.0 / (s * nn))
        rs = jnp.sum(a, axis=1, keepdims=True)
        a = a * (1.0 / (rs + 1e-8))
        a_scr[rows, :] = a
        cs_scr[0:1] += jnp.sum(a, axis=0, keepdims=True)

    def _mid(r):
        @pl.when(i == 0)
        def _():
            cs_scr[r + 1:r + 2] = jnp.zeros_like(cs_scr[0:1])

        a = a_scr[rows, :] * (1.0 / (cs_scr[r:r + 1] + 1e-8))
        rs = jnp.sum(a, axis=1, keepdims=True)
        a = a * (1.0 / (rs + 1e-8))
        a_scr[rows, :] = a
        cs_scr[r + 1:r + 2] += jnp.sum(a, axis=0, keepdims=True)

    @pl.when(p == 1)
    def _p1():
        _mid(0)

    @pl.when(p == 2)
    def _p2():
        _mid(1)

    @pl.when(p == 3)
    def _p3():
        cs3 = cs_scr[2:3]

        @pl.when(i == 0)
        def _():
            align_ref[0, 0] = jnp.float32(0.0)
            spars_ref[0, 0] = jnp.float32(0.0)
            c3 = cs3[0]
            q = c3 / (c3 + 1e-8)
            marg_ref[0, 0] = jnp.sum(q * (jnp.log(q + 1e-12) - (1.0 / nn)))
            bad_ref[0, 0] = jnp.sum((c3 == 0.0).astype(F32))

        a = a_scr[rows, :] * (1.0 / (cs3 + 1e-8))
        rsa = jnp.sum(a, axis=1, keepdims=True)
        cxy_ref[...] = (a * (1.0 / rsa)).astype(BF16)
        csa = cs3 / (cs3 + 1e-8)
        cyxt_ref[...] = (a * (1.0 / csa)).astype(BF16)
        bad_ref[0, 0] += jnp.sum((rsa == 0.0).astype(F32))
        d2 = (sn0_ref[0, 0][:, None] + sn1_ref[0, 0][None, :]
              - 2.0 * _dot(h0_ref[0], h1_ref[0], ((1,), (1,)), HI))
        cd = jnp.sqrt(jnp.maximum(d2, 0.0))
        align_ref[0, 0] += jnp.sum(a * cd)
        spars_ref[0, 0] += jnp.sum(a * jnp.log(a + 1e-10))


def _t1_body(d_ref, c_ref, o_ref):
    o_ref[...] = _dot(d_ref[...].astype(BF16), c_ref[...],
                      ((1,), (1,))).astype(BF16)


def _t2_body(d_ref, c_ref, o_ref):
    o_ref[...] = _dot(d_ref[...].astype(BF16), c_ref[...],
                      ((1,), (0,))).astype(BF16)


def _p1_body(c_ref, t_ref, d0_ref, f1_ref, f0_ref, ssq_ref, mse_ref):
    @pl.when(pl.program_id(0) == 0)
    def _init():
        ssq_ref[0, 0] = jnp.float32(0.0)
        mse_ref[0, 0] = jnp.float32(0.0)

    p = _dot(c_ref[...], t_ref[...], ((1,), (0,)))
    d = d0_ref[...] - p
    ssq_ref[0, 0] += jnp.sum(d * d)
    e = f0_ref[...] - _dot(c_ref[...], f1_ref[...], ((1,), (0,)))
    mse_ref[0, 0] += jnp.sum(e * e)


def _p2_body(ct_ref, t_ref, d1_ref, f0_ref, f1_ref, ssq_ref, mse_ref):
    @pl.when(pl.program_id(0) == 0)
    def _init():
        ssq_ref[0, 0] = jnp.float32(0.0)
        mse_ref[0, 0] = jnp.float32(0.0)

    p = _dot(ct_ref[...], t_ref[...], ((0,), (0,)))
    d = d1_ref[...] - p
    ssq_ref[0, 0] += jnp.sum(d * d)
    e = f1_ref[...] - _dot(ct_ref[...], f0_ref[...], ((0,), (0,)))
    mse_ref[0, 0] += jnp.sum(e * e)


def _scalar_spec():
    return pl.BlockSpec(memory_space=pltpu.SMEM)


def _heavy_losses(cxy_bf, cyxt_bf, f0, f1, d0, d1):
    n, fd = f0.shape
    bm = min(512, n)
    grid = (n // bm,)
    f0_bf = f0.astype(BF16)
    f1_bf = f1.astype(BF16)

    t1 = pl.pallas_call(
        _t1_body, grid=grid,
        in_specs=[pl.BlockSpec((bm, n), lambda i: (i, 0)),
                  pl.BlockSpec((n, n), lambda i: (0, 0))],
        out_specs=pl.BlockSpec((bm, n), lambda i: (i, 0)),
        out_shape=jax.ShapeDtypeStruct((n, n), BF16))(d1, cxy_bf)

    ssq1, mse1 = pl.pallas_call(
        _p1_body, grid=grid,
        in_specs=[pl.BlockSpec((bm, n), lambda i: (i, 0)),
                  pl.BlockSpec((n, n), lambda i: (0, 0)),
                  pl.BlockSpec((bm, n), lambda i: (i, 0)),
                  pl.BlockSpec((n, fd), lambda i: (0, 0)),
                  pl.BlockSpec((bm, fd), lambda i: (i, 0))],
        out_specs=[_scalar_spec(), _scalar_spec()],
        out_shape=[jax.ShapeDtypeStruct((1, 1), F32),
                   jax.ShapeDtypeStruct((1, 1), F32)])(
        cxy_bf, t1, d0, f1_bf, f0)

    t2 = pl.pallas_call(
        _t2_body, grid=grid,
        in_specs=[pl.BlockSpec((bm, n), lambda i: (i, 0)),
                  pl.BlockSpec((n, n), lambda i: (0, 0))],
        out_specs=pl.BlockSpec((bm, n), lambda i: (i, 0)),
        out_shape=jax.ShapeDtypeStruct((n, n), BF16))(d0, cyxt_bf)

    ssq2, mse2 = pl.pallas_call(
        _p2_body, grid=grid,
        in_specs=[pl.BlockSpec((n, bm), lambda i: (0, i)),
                  pl.BlockSpec((n, n), lambda i: (0, 0)),
                  pl.BlockSpec((bm, n), lambda i: (i, 0)),
                  pl.BlockSpec((n, fd), lambda i: (0, 0)),
                  pl.BlockSpec((bm, fd), lambda i: (i, 0))],
        out_specs=[_scalar_spec(), _scalar_spec()],
        out_shape=[jax.ShapeDtypeStruct((1, 1), F32),
                   jax.ShapeDtypeStruct((1, 1), F32)])(
        cyxt_bf, t2, d1, f0_bf, f1)

    la_fix = (mse1[0, 0] + mse2[0, 0]) / jnp.float32(n * fd)
    l_maint = (jnp.sqrt(ssq1[0, 0]) + jnp.sqrt(ssq2[0, 0])) / jnp.float32(n)
    return la_fix, l_maint


def kernel(feature_set, spot_feature_set, adj_set, graph_neigh_set,
           label_CSL_set, dist_mat_set, enc_w, dec_w, disc_w, disc_b, Ms):
    del spot_feature_set
    kb, n, fd = feature_set.shape
    lt = enc_w.shape[-1]
    br = min(512, n)
    ni = n // br
    brp = min(512, n)
    nip = n // brp
    m0 = Ms[0]
    b2 = jnp.reshape(disc_b, (1, 1)).astype(F32)

    xe = pl.pallas_call(
        _xe_body, grid=(kb,),
        in_specs=[pl.BlockSpec((1, n, fd), lambda k: (k, 0, 0)),
                  pl.BlockSpec((1, fd, lt), lambda k: (k, 0, 0))],
        out_specs=pl.BlockSpec((1, n, lt), lambda k: (k, 0, 0)),
        out_shape=jax.ShapeDtypeStruct((kb, n, lt), F32))(feature_set, enc_w)
    perm = jax.random.permutation(jax.random.key(42), n)
    xea = jnp.take(xe, perm, axis=1)

    ni1 = ni - 1

    def _ph0_in(k, ph, i):
        return (k, jnp.where(ph == 0, i, ni1), 0)

    def _ph1_in(k, ph, i):
        return (k, jnp.where(ph == 1, i, 0), 0)

    h, sn, sl_raw, ft_raw = pl.pallas_call(
        functools.partial(_enc_body, br=br),
        grid=(kb, 2, ni),
        in_specs=[pl.BlockSpec((1, br, n), _ph0_in),
                  pl.BlockSpec((1, br, n), _ph1_in),
                  pl.BlockSpec((1, br, fd), _ph1_in),
                  pl.BlockSpec((1, br, 2), _ph1_in),
                  pl.BlockSpec((1, n, lt), lambda k, ph, i: (k, 0, 0)),
                  pl.BlockSpec((1, n, lt), lambda k, ph, i: (k, 0, 0)),
                  pl.BlockSpec((1, lt, fd), lambda k, ph, i: (k, 0, 0)),
                  pl.BlockSpec((lt, lt), lambda k, ph, i: (0, 0)),
                  _scalar_spec()],
        out_specs=[pl.BlockSpec((1, br, lt), _ph0_in),
                   pl.BlockSpec((1, 1, br),
                                lambda k, ph, i:
                                (k, 0, jnp.where(ph == 0, i, ni1))),
                   _scalar_spec(), _scalar_spec()],
        out_shape=[jax.ShapeDtypeStruct((kb, n, lt), F32),
                   jax.ShapeDtypeStruct((kb, 1, n), F32),
                   jax.ShapeDtypeStruct((1, 1), F32),
                   jax.ShapeDtypeStruct((1, 1), F32)],
        scratch_shapes=[pltpu.VMEM((n, n), BF16),
                        pltpu.VMEM((n, lt), BF16),
                        pltpu.VMEM((n, lt), BF16),
                        pltpu.VMEM((n, lt), BF16)])(
        adj_set, graph_neigh_set, feature_set, label_CSL_set, xe, xea,
        dec_w, disc_w, b2)

    def _out_map(p, i):
        return (jnp.where(p == 3, i, 0), 0)

    cxy_bf, cyxt_bf, align_raw, spars_raw, marg_raw, bad = pl.pallas_call(
        functools.partial(_sink_body, nn=float(n), lt=lt, brp=brp),
        grid=(4, nip),
        in_specs=[pl.BlockSpec((lt, lt), lambda p, i: (0, 0)),
                  pl.BlockSpec((1, brp, lt), lambda p, i: (0, i, 0)),
                  pl.BlockSpec((1, n, lt), lambda p, i: (1, 0, 0)),
                  pl.BlockSpec((1, 1, brp), lambda p, i: (0, 0, i)),
                  pl.BlockSpec((1, 1, n), lambda p, i: (1, 0, 0))],
        out_specs=[pl.BlockSpec((brp, n), _out_map),
                   pl.BlockSpec((brp, n), _out_map),
                   _scalar_spec(), _scalar_spec(), _scalar_spec(),
                   _scalar_spec()],
        out_shape=[jax.ShapeDtypeStruct((n, n), BF16),
                   jax.ShapeDtypeStruct((n, n), BF16),
                   jax.ShapeDtypeStruct((1, 1), F32),
                   jax.ShapeDtypeStruct((1, 1), F32),
                   jax.ShapeDtypeStruct((1, 1), F32),
                   jax.ShapeDtypeStruct((1, 1), F32)],
        scratch_shapes=[pltpu.VMEM((n, n), F32),
                        pltpu.VMEM((8, n), F32),
                        pltpu.VMEM((n, lt), F32)])(
        m0, h, h, sn, sn)

    f0 = feature_set[0]
    f1 = feature_set[1]
    d0 = dist_mat_set[0]
    d1 = dist_mat_set[1]

    la_fix, l_maint = lax.cond(
        bad[0, 0] > 0.0,
        lambda ops: (jnp.float32(jnp.nan), jnp.float32(jnp.nan)),
        lambda ops: _heavy_losses(*ops),
        (cxy_bf, cyxt_bf, f0, f1, d0, d1))

    nf = jnp.float32(n)
    loss_sl = sl_raw[0, 0] / (nf * 2.0)
    loss_feat = ft_raw[0, 0] / (nf * jnp.float32(fd))
    loss_align = align_raw[0, 0]
    kl_pq = marg_raw[0, 0] / nf
    p = 1.0 / nf
    kl_pp = p * (jnp.log(p + 1e-12) - p)
    loss_marginal = (kl_pq - kl_pp) * nf
    loss_sparsity = -spars_raw[0, 0]
    return jnp.stack([loss_sl, loss_feat, loss_align, la_fix, l_maint,
                      loss_marginal, loss_sparsity])
